# baseline (speedup 1.0000x reference)
import functools

import numpy as np
import jax
import jax.numpy as jnp
from jax import lax
from jax.experimental import pallas as pl
from jax.experimental.pallas import tpu as pltpu
from jax.experimental.pallas import tpu_sc as plsc

_B = 4096
_D = 32
_S = 25
_W = 128
_RPL = _W // _D
_NC = 2
_NS = 16
_NW = _NC * _NS
_L = 16
_BPW = _B // _NW
_OPW = _BPW * _S
_NV = _OPW // _L

_PERM = np.asarray(
    [31, 7, 4, 29, 16, 19, 2, 5, 30, 3, 22, 6, 18, 10, 11, 15,
     20, 8, 24, 9, 25, 13, 14, 17, 23, 0, 21, 26, 1, 28, 27, 12],
    dtype=np.int32,
)

_B_IDX = np.repeat(np.arange(_BPW, dtype=np.int32), _S)
_R_IDX = np.tile(np.arange(_S, dtype=np.int32), _BPW)

_mesh = plsc.VectorSubcoreMesh(core_axis_name="c", subcore_axis_name="s")


@functools.partial(
    pl.kernel,
    out_type=jax.ShapeDtypeStruct((_B * _S,), jnp.int32),
    mesh=_mesh,
    compiler_params=pltpu.CompilerParams(needs_layout_passes=False),
    scratch_types=[
        pltpu.VMEM((_BPW,), jnp.int32),
        pltpu.VMEM((_BPW, _W), jnp.int32),
        pltpu.VMEM((_OPW,), jnp.int32),
        pltpu.VMEM((_OPW,), jnp.int32),
        pltpu.VMEM((_OPW,), jnp.int32),
        pltpu.SemaphoreType.DMA,
    ],
)
def _sample_sc(table_hbm, qids_hbm, bi_hbm, ci_hbm, out_hbm,
               ids_v, rows_v, out_v, bi_v, ci_v, sem):
    wid = lax.axis_index("s") * _NC + lax.axis_index("c")
    base = wid * _BPW
    obase = wid * _OPW
    pltpu.sync_copy(bi_hbm, bi_v)
    pltpu.sync_copy(ci_hbm.at[pl.ds(obase, _OPW)], ci_v)
    pltpu.sync_copy(qids_hbm.at[pl.ds(base, _BPW)], ids_v)
    pltpu.async_copy(table_hbm.at[ids_v], rows_v, sem).wait()

    def body(v, carry):
        bvec = bi_v[pl.ds(v * _L, _L)]
        cvec = ci_v[pl.ds(v * _L, _L)]
        out_v[pl.ds(v * _L, _L)] = plsc.load_gather(rows_v, [bvec, cvec])
        return carry

    lax.fori_loop(0, _NV, body, 0)
    pltpu.sync_copy(out_v, out_hbm.at[pl.ds(obase, _OPW)])


def kernel(ids, num_samples, t, adj_info):
    T, N, D = adj_info.shape
    table = adj_info.reshape(T * N // _RPL, _W)
    rid = ids + t * N
    qids = (rid // _RPL).astype(jnp.int32)
    cols = lax.dynamic_slice(jnp.asarray(_PERM), (num_samples - _S,), (_S,))
    ci = ((rid % _RPL)[:, None] * _D + cols[None, :]).reshape(-1).astype(jnp.int32)
    bi = jnp.asarray(_B_IDX)
    out = _sample_sc(table, qids, bi, ci)
    return out.reshape(_B, _S)


# speedup vs baseline: 4.1012x; 4.1012x over previous
import functools

import numpy as np
import jax
import jax.numpy as jnp
from jax import lax
from jax.experimental import pallas as pl
from jax.experimental.pallas import tpu as pltpu
from jax.experimental.pallas import tpu_sc as plsc

_B = 4096
_D = 32
_S = 25
_N = 100000
_NC = 2
_NS = 16
_NW = _NC * _NS
_L = 16
_NVQ = _B // _L

_PERM = np.asarray(
    [31, 7, 4, 29, 16, 19, 2, 5, 30, 3, 22, 6, 18, 10, 11, 15,
     20, 8, 24, 9, 25, 13, 14, 17, 23, 0, 21, 26, 1, 28, 27, 12],
    dtype=np.int32,
)

_mesh = plsc.VectorSubcoreMesh(core_axis_name="c", subcore_axis_name="s")


@functools.partial(
    pl.kernel,
    out_type=jax.ShapeDtypeStruct((_S * _B,), jnp.int32),
    mesh=_mesh,
    compiler_params=pltpu.CompilerParams(needs_layout_passes=False),
    scratch_types=[
        pltpu.VMEM((_NW + _L,), jnp.int32),
        pltpu.VMEM((_B,), jnp.int32),
        pltpu.VMEM((_N,), jnp.int32),
        pltpu.VMEM((_B,), jnp.int32),
    ],
)
def _sample_sc(tableT_hbm, ids_hbm, rj_hbm, out_hbm, rj_s, ids_v, row_v, gat_v):
    wid = lax.axis_index("s") * _NC + lax.axis_index("c")
    pltpu.sync_copy(rj_hbm, rj_s)

    @pl.when(wid < _S)
    def _():
        r = rj_s[pl.ds(wid, _L)][0]
        pltpu.sync_copy(ids_hbm, ids_v)
        pltpu.sync_copy(tableT_hbm.at[r], row_v)

        def body(v, carry):
            ivec = ids_v[pl.ds(v * _L, _L)]
            gat_v[pl.ds(v * _L, _L)] = plsc.load_gather(row_v, [ivec])
            return carry

        lax.fori_loop(0, _NVQ, body, 0)
        pltpu.sync_copy(gat_v, out_hbm.at[pl.ds(wid * _B, _B)])


def kernel(ids, num_samples, t, adj_info):
    T, N, D = adj_info.shape
    tableT = jnp.swapaxes(adj_info, 1, 2).reshape(T * D, N)
    cols = lax.dynamic_slice(jnp.asarray(_PERM), (num_samples - _S,), (_S,))
    rj = t * D + jnp.concatenate([cols, jnp.zeros((_NW + _L - _S,), jnp.int32)])
    out = _sample_sc(tableT, ids, rj)
    return out.reshape(_S, _B).T


# speedup vs baseline: 4.3651x; 1.0644x over previous
import functools

import numpy as np
import jax
import jax.numpy as jnp
from jax import lax
from jax.experimental import pallas as pl
from jax.experimental.pallas import tpu as pltpu
from jax.experimental.pallas import tpu_sc as plsc

_B = 4096
_D = 32
_S = 25
_N = 100000
_NC = 2
_NS = 16
_NW = _NC * _NS
_L = 16
_NVQ = _B // _L

_PERM = np.zeros(_D + _L + _NW - _S, dtype=np.int32)
_PERM[:_D] = [31, 7, 4, 29, 16, 19, 2, 5, 30, 3, 22, 6, 18, 10, 11, 15,
              20, 8, 24, 9, 25, 13, 14, 17, 23, 0, 21, 26, 1, 28, 27, 12]

_mesh = plsc.VectorSubcoreMesh(core_axis_name="c", subcore_axis_name="s")


@functools.partial(
    pl.kernel,
    out_type=jax.ShapeDtypeStruct((_S, _B), jnp.int32),
    mesh=_mesh,
    compiler_params=pltpu.CompilerParams(needs_layout_passes=False),
    scratch_types=[
        pltpu.VMEM((_L,), jnp.int32),
        pltpu.VMEM((_PERM.size,), jnp.int32),
        pltpu.VMEM((_B,), jnp.int32),
        pltpu.VMEM((_N,), jnp.int32),
        pltpu.VMEM((_B,), jnp.int32),
    ],
)
def _sample_sc(tableT_hbm, ids_hbm, scal_hbm, perm_hbm, out_hbm,
               scal_v, perm_v, ids_v, row_v, gat_v):
    wid = lax.axis_index("s") * _NC + lax.axis_index("c")

    @pl.when(wid < _S)
    def _():
        pltpu.sync_copy(scal_hbm, scal_v)
        pltpu.sync_copy(perm_hbm, perm_v)
        pltpu.sync_copy(ids_hbm, ids_v)
        sv = scal_v[pl.ds(0, _L)]
        pv = perm_v[pl.ds(sv[0] - _S + wid, _L)]
        r = sv[1] * _D + pv[0]
        pltpu.sync_copy(tableT_hbm.at[r], row_v)

        def body(v, carry):
            ivec = ids_v[pl.ds(v * _L, _L)]
            gat_v[pl.ds(v * _L, _L)] = plsc.load_gather(row_v, [ivec])
            return carry

        lax.fori_loop(0, _NVQ, body, 0)
        pltpu.sync_copy(gat_v, out_hbm.at[wid])


def kernel(ids, num_samples, t, adj_info):
    T, N, D = adj_info.shape
    tableT = jnp.swapaxes(adj_info, 1, 2).reshape(T * D, N)
    scal = jnp.zeros((_L,), jnp.int32).at[0].set(num_samples).at[1].set(t)
    out = _sample_sc(tableT, ids, scal, jnp.asarray(_PERM))
    return out.T


# speedup vs baseline: 4.6547x; 1.0663x over previous
import functools

import numpy as np
import jax
import jax.numpy as jnp
from jax import lax
from jax.experimental import pallas as pl
from jax.experimental.pallas import tpu as pltpu
from jax.experimental.pallas import tpu_sc as plsc

_B = 4096
_D = 32
_S = 25
_N = 100000
_NC = 2
_NS = 16
_NW = _NC * _NS
_L = 16
_NVQ = _B // _L
_UNROLL = 8

_PERM = np.asarray(
    [31, 7, 4, 29, 16, 19, 2, 5, 30, 3, 22, 6, 18, 10, 11, 15,
     20, 8, 24, 9, 25, 13, 14, 17, 23, 0, 21, 26, 1, 28, 27, 12],
    dtype=np.int32,
)
_CTRL_LEN = _L + _D + _L + (_NW - _S)

_mesh = plsc.VectorSubcoreMesh(core_axis_name="c", subcore_axis_name="s")


@functools.partial(
    pl.kernel,
    out_type=jax.ShapeDtypeStruct((_S, _B), jnp.int32),
    mesh=_mesh,
    compiler_params=pltpu.CompilerParams(
        needs_layout_passes=False,
        disable_bounds_checks=True,
        disable_semaphore_checks=True,
    ),
    scratch_types=[
        pltpu.VMEM((_CTRL_LEN,), jnp.int32),
        pltpu.VMEM((_B,), jnp.int32),
        pltpu.VMEM((_N,), jnp.int32),
        pltpu.VMEM((_B,), jnp.int32),
        pltpu.SemaphoreType.DMA,
        pltpu.SemaphoreType.DMA,
    ],
)
def _sample_sc(tableT_hbm, ids_hbm, ctrl_hbm, out_hbm,
               ctrl_v, ids_v, row_v, gat_v, sem_ids, sem_row):
    wid = lax.axis_index("s") * _NC + lax.axis_index("c")

    @pl.when(wid < _S)
    def _():
        a_ids = pltpu.async_copy(ids_hbm, ids_v, sem_ids)
        pltpu.sync_copy(ctrl_hbm, ctrl_v)
        sv = ctrl_v[pl.ds(0, _L)]
        pv = ctrl_v[pl.ds(_L + sv[0] - _S + wid, _L)]
        r = sv[1] * _D + pv[0]
        pltpu.async_copy(tableT_hbm.at[r], row_v, sem_row).wait()
        a_ids.wait()

        def body(v, carry):
            base = v * (_UNROLL * _L)
            for u in range(_UNROLL):
                ivec = ids_v[pl.ds(base + u * _L, _L)]
                gat_v[pl.ds(base + u * _L, _L)] = plsc.load_gather(row_v, [ivec])
            return carry

        lax.fori_loop(0, _NVQ // _UNROLL, body, 0)
        pltpu.sync_copy(gat_v, out_hbm.at[wid])


def kernel(ids, num_samples, t, adj_info):
    T, N, D = adj_info.shape
    tableT = jnp.swapaxes(adj_info, 1, 2).reshape(T * D, N)
    ctrl = (
        jnp.zeros((_CTRL_LEN,), jnp.int32)
        .at[0].set(num_samples)
        .at[1].set(t)
        .at[_L : _L + _D].set(jnp.asarray(_PERM))
    )
    out = _sample_sc(tableT, ids, ctrl)
    return out.T
